# baseline (speedup 1.0000x reference)
import functools

import jax
import jax.numpy as jnp
from jax.experimental import pallas as pl
from jax.experimental.pallas import tpu as pltpu

_LOG2E = 1.4426950408889634
_FAUG = 384


def _proj_kernel(x_ref, wv_ref, bv_ref, wq_ref, bq_ref, wk_ref, bk_ref,
                 xvb_ref, el_ref, er_ref, elmax_ref):
    i = pl.program_id(0)
    xv = jax.lax.dot_general(
        x_ref[...], wv_ref[...], (((1,), (1,)), ((), ())),
        preferred_element_type=jnp.float32) + bv_ref[...]
    xvb_ref[:, :xv.shape[1]] = xv.astype(jnp.bfloat16)
    lane = jax.lax.broadcasted_iota(
        jnp.int32, (xv.shape[0], _FAUG - xv.shape[1]), 1)
    xvb_ref[:, xv.shape[1]:] = (lane == 0).astype(jnp.bfloat16)
    el = jnp.sum(xv * wq_ref[...], axis=1, keepdims=True) + bq_ref[0, 0]
    el_ref[...] = el
    er_col = jnp.sum(xv * wk_ref[...], axis=1, keepdims=True) + bk_ref[0, 0]
    er_ref[...] = er_col.T
    bmax = jnp.max(el, keepdims=True)

    @pl.when(i == 0)
    def _():
        elmax_ref[...] = bmax

    @pl.when(i > 0)
    def _():
        elmax_ref[...] = jnp.maximum(elmax_ref[...], bmax)


def _attn_kernel(adj_ref, el_ref, er_ref, xvb_ref, elmax_ref, out_ref,
                 acc_ref, *, ni, f):
    i = pl.program_id(1)

    @pl.when(i == 0)
    def _():
        acc_ref[...] = jnp.zeros_like(acc_ref)

    el = el_ref[...]
    er = er_ref[...]
    mtop = elmax_ref[...] + er
    mj = jnp.maximum(mtop, 0.2 * mtop)
    a = (er - mj) * _LOG2E
    b = (0.2 * er - mj) * _LOG2E
    elc = el * _LOG2E
    elc2 = elc * 0.2
    z = jnp.maximum(elc + a, elc2 + b)
    p = (jnp.exp2(z) * adj_ref[...].astype(jnp.float32)).astype(jnp.bfloat16)
    acc_ref[...] += jax.lax.dot_general(
        p, xvb_ref[...], (((0,), (0,)), ((), ())),
        preferred_element_type=jnp.float32)

    @pl.when(i == ni - 1)
    def _():
        d = jnp.maximum(acc_ref[:, f:f + 1], 1e-20)
        out_ref[...] = acc_ref[:, :f] * (1.0 / d)


def kernel(x, adj, Wv, bv, wq, bq, wk, bk):
    n, _ = x.shape
    f = Wv.shape[0]

    bi1 = min(512, n)
    ni1 = n // bi1
    xvb, el, er, elmax = pl.pallas_call(
        _proj_kernel,
        grid=(ni1,),
        in_specs=[
            pl.BlockSpec((bi1, x.shape[1]), lambda i: (i, 0)),
            pl.BlockSpec(Wv.shape, lambda i: (0, 0)),
            pl.BlockSpec((1, f), lambda i: (0, 0)),
            pl.BlockSpec((1, f), lambda i: (0, 0)),
            pl.BlockSpec((1, 1), lambda i: (0, 0)),
            pl.BlockSpec((1, f), lambda i: (0, 0)),
            pl.BlockSpec((1, 1), lambda i: (0, 0)),
        ],
        out_specs=[
            pl.BlockSpec((bi1, _FAUG), lambda i: (i, 0)),
            pl.BlockSpec((bi1, 1), lambda i: (i, 0)),
            pl.BlockSpec((1, bi1), lambda i: (0, i)),
            pl.BlockSpec((1, 1), lambda i: (0, 0)),
        ],
        out_shape=[
            jax.ShapeDtypeStruct((n, _FAUG), jnp.bfloat16),
            jax.ShapeDtypeStruct((n, 1), jnp.float32),
            jax.ShapeDtypeStruct((1, n), jnp.float32),
            jax.ShapeDtypeStruct((1, 1), jnp.float32),
        ],
        compiler_params=pltpu.CompilerParams(
            dimension_semantics=("arbitrary",)),
    )(x, Wv, bv.reshape(1, f), wq, bq.reshape(1, 1), wk, bk.reshape(1, 1))

    bi = min(2048, n)
    bj = min(512, n)
    ni = n // bi
    nj = n // bj
    out = pl.pallas_call(
        functools.partial(_attn_kernel, ni=ni, f=f),
        grid=(nj, ni),
        in_specs=[
            pl.BlockSpec((bi, bj), lambda j, i: (i, j)),
            pl.BlockSpec((bi, 1), lambda j, i: (i, 0)),
            pl.BlockSpec((1, bj), lambda j, i: (0, j)),
            pl.BlockSpec((bi, _FAUG), lambda j, i: (i, 0)),
            pl.BlockSpec((1, 1), lambda j, i: (0, 0)),
        ],
        out_specs=pl.BlockSpec((bj, f), lambda j, i: (j, 0)),
        out_shape=jax.ShapeDtypeStruct((n, f), jnp.float32),
        scratch_shapes=[
            pltpu.VMEM((bj, _FAUG), jnp.float32),
        ],
        compiler_params=pltpu.CompilerParams(
            dimension_semantics=("parallel", "arbitrary")),
    )(adj, el, er, xvb, elmax)
    return out


# speedup vs baseline: 1.2321x; 1.2321x over previous
import functools

import jax
import jax.numpy as jnp
from jax.experimental import pallas as pl
from jax.experimental.pallas import tpu as pltpu

_LOG2E = 1.4426950408889634
_FAUG = 384
_BI = 2048
_BJ = 1024


def _proj_kernel(x_ref, wv_ref, bv_ref, wq_ref, bq_ref, wk_ref, bk_ref,
                 xvb_ref, el_ref, er_ref, elmax_ref):
    i = pl.program_id(0)
    xv = jax.lax.dot_general(
        x_ref[...], wv_ref[...], (((1,), (1,)), ((), ())),
        preferred_element_type=jnp.float32) + bv_ref[...]
    xvb_ref[:, :xv.shape[1]] = xv.astype(jnp.bfloat16)
    lane = jax.lax.broadcasted_iota(
        jnp.int32, (xv.shape[0], _FAUG - xv.shape[1]), 1)
    xvb_ref[:, xv.shape[1]:] = (lane == 0).astype(jnp.bfloat16)
    el = jnp.sum(xv * wq_ref[...], axis=1, keepdims=True) + bq_ref[0, 0]
    el_ref[...] = el
    er_col = jnp.sum(xv * wk_ref[...], axis=1, keepdims=True) + bk_ref[0, 0]
    er_ref[...] = er_col.T
    bmax = jnp.max(el, keepdims=True)

    @pl.when(i == 0)
    def _():
        elmax_ref[...] = bmax

    @pl.when(i > 0)
    def _():
        elmax_ref[...] = jnp.maximum(elmax_ref[...], bmax)


def _attn_kernel(adj_ref, el_ref, er_ref, xvb_ref, elmax_ref, out_ref,
                 acc_ref, *, ni, f):
    i = pl.program_id(1)

    @pl.when(i == 0)
    def _():
        acc_ref[...] = jnp.zeros_like(acc_ref)

    el = el_ref[...]
    er = er_ref[...]
    mtop = elmax_ref[...] + er
    mj = jnp.maximum(mtop, 0.2 * mtop)
    a = (er - mj) * _LOG2E
    b = (0.2 * er - mj) * _LOG2E
    elc = el * _LOG2E
    elc2 = elc * 0.2
    z = jnp.maximum(elc + a, elc2 + b)
    p = (jnp.exp2(z) * adj_ref[...].astype(jnp.float32)).astype(jnp.bfloat16)
    acc_ref[...] += jax.lax.dot_general(
        p, xvb_ref[...], (((0,), (0,)), ((), ())),
        preferred_element_type=jnp.float32)

    @pl.when(i == ni - 1)
    def _():
        d = jnp.maximum(acc_ref[:, f:f + 1], 1e-20)
        out_ref[...] = acc_ref[:, :f] * (1.0 / d)


def kernel(x, adj, Wv, bv, wq, bq, wk, bk):
    n, _ = x.shape
    f = Wv.shape[0]

    bi1 = min(512, n)
    ni1 = n // bi1
    xvb, el, er, elmax = pl.pallas_call(
        _proj_kernel,
        grid=(ni1,),
        in_specs=[
            pl.BlockSpec((bi1, x.shape[1]), lambda i: (i, 0)),
            pl.BlockSpec(Wv.shape, lambda i: (0, 0)),
            pl.BlockSpec((1, f), lambda i: (0, 0)),
            pl.BlockSpec((1, f), lambda i: (0, 0)),
            pl.BlockSpec((1, 1), lambda i: (0, 0)),
            pl.BlockSpec((1, f), lambda i: (0, 0)),
            pl.BlockSpec((1, 1), lambda i: (0, 0)),
        ],
        out_specs=[
            pl.BlockSpec((bi1, _FAUG), lambda i: (i, 0)),
            pl.BlockSpec((bi1, 1), lambda i: (i, 0)),
            pl.BlockSpec((1, bi1), lambda i: (0, i)),
            pl.BlockSpec((1, 1), lambda i: (0, 0)),
        ],
        out_shape=[
            jax.ShapeDtypeStruct((n, _FAUG), jnp.bfloat16),
            jax.ShapeDtypeStruct((n, 1), jnp.float32),
            jax.ShapeDtypeStruct((1, n), jnp.float32),
            jax.ShapeDtypeStruct((1, 1), jnp.float32),
        ],
        compiler_params=pltpu.CompilerParams(
            dimension_semantics=("arbitrary",)),
    )(x, Wv, bv.reshape(1, f), wq, bq.reshape(1, 1), wk, bk.reshape(1, 1))

    bi = min(_BI, n)
    bj = min(_BJ, n)
    ni = n // bi
    nj = n // bj
    out = pl.pallas_call(
        functools.partial(_attn_kernel, ni=ni, f=f),
        grid=(nj, ni),
        in_specs=[
            pl.BlockSpec((bi, bj), lambda j, i: (i, j)),
            pl.BlockSpec((bi, 1), lambda j, i: (i, 0)),
            pl.BlockSpec((1, bj), lambda j, i: (0, j)),
            pl.BlockSpec((bi, _FAUG), lambda j, i: (i, 0)),
            pl.BlockSpec((1, 1), lambda j, i: (0, 0)),
        ],
        out_specs=pl.BlockSpec((bj, f), lambda j, i: (j, 0)),
        out_shape=jax.ShapeDtypeStruct((n, f), jnp.float32),
        scratch_shapes=[
            pltpu.VMEM((bj, _FAUG), jnp.float32),
        ],
        compiler_params=pltpu.CompilerParams(
            dimension_semantics=("parallel", "arbitrary")),
    )(adj, el, er, xvb, elmax)
    return out


# speedup vs baseline: 1.3176x; 1.0694x over previous
import functools

import jax
import jax.numpy as jnp
from jax.experimental import pallas as pl
from jax.experimental.pallas import tpu as pltpu

_LOG2E = 1.4426950408889634
_FAUG = 384
_BI = 1024
_BJ = 2048


def _proj_kernel(x_ref, wv_ref, bv_ref, wq_ref, bq_ref, wk_ref, bk_ref,
                 xvb_ref, el_ref, er_ref, elmax_ref):
    i = pl.program_id(0)
    xv = jax.lax.dot_general(
        x_ref[...], wv_ref[...], (((1,), (1,)), ((), ())),
        preferred_element_type=jnp.float32) + bv_ref[...]
    xvb_ref[:, :xv.shape[1]] = xv.astype(jnp.bfloat16)
    lane = jax.lax.broadcasted_iota(
        jnp.int32, (xv.shape[0], _FAUG - xv.shape[1]), 1)
    xvb_ref[:, xv.shape[1]:] = (lane == 0).astype(jnp.bfloat16)
    el = jnp.sum(xv * wq_ref[...], axis=1, keepdims=True) + bq_ref[0, 0]
    el_ref[...] = el
    er_col = jnp.sum(xv * wk_ref[...], axis=1, keepdims=True) + bk_ref[0, 0]
    er_ref[...] = er_col.T
    bmax = jnp.max(el, keepdims=True)

    @pl.when(i == 0)
    def _():
        elmax_ref[...] = bmax

    @pl.when(i > 0)
    def _():
        elmax_ref[...] = jnp.maximum(elmax_ref[...], bmax)


def _attn_kernel(adj_ref, el_ref, er_ref, xvb_ref, elmax_ref, out_ref,
                 acc_ref, *, ni, f):
    i = pl.program_id(1)

    @pl.when(i == 0)
    def _():
        acc_ref[...] = jnp.zeros_like(acc_ref)

    el = el_ref[...]
    er = er_ref[...]
    mtop = elmax_ref[...] + er
    mj = jnp.maximum(mtop, 0.2 * mtop)
    a = (er - mj) * _LOG2E
    b = (0.2 * er - mj) * _LOG2E
    elc = el * _LOG2E
    elc2 = elc * 0.2
    z = jnp.maximum(elc + a, elc2 + b)
    p = (jnp.exp2(z) * adj_ref[...].astype(jnp.float32)).astype(jnp.bfloat16)
    acc_ref[...] += jax.lax.dot_general(
        p, xvb_ref[...], (((0,), (0,)), ((), ())),
        preferred_element_type=jnp.float32)

    @pl.when(i == ni - 1)
    def _():
        d = jnp.maximum(acc_ref[:, f:f + 1], 1e-20)
        out_ref[...] = acc_ref[:, :f] * (1.0 / d)


def kernel(x, adj, Wv, bv, wq, bq, wk, bk):
    n, _ = x.shape
    f = Wv.shape[0]

    bi1 = min(512, n)
    ni1 = n // bi1
    xvb, el, er, elmax = pl.pallas_call(
        _proj_kernel,
        grid=(ni1,),
        in_specs=[
            pl.BlockSpec((bi1, x.shape[1]), lambda i: (i, 0)),
            pl.BlockSpec(Wv.shape, lambda i: (0, 0)),
            pl.BlockSpec((1, f), lambda i: (0, 0)),
            pl.BlockSpec((1, f), lambda i: (0, 0)),
            pl.BlockSpec((1, 1), lambda i: (0, 0)),
            pl.BlockSpec((1, f), lambda i: (0, 0)),
            pl.BlockSpec((1, 1), lambda i: (0, 0)),
        ],
        out_specs=[
            pl.BlockSpec((bi1, _FAUG), lambda i: (i, 0)),
            pl.BlockSpec((bi1, 1), lambda i: (i, 0)),
            pl.BlockSpec((1, bi1), lambda i: (0, i)),
            pl.BlockSpec((1, 1), lambda i: (0, 0)),
        ],
        out_shape=[
            jax.ShapeDtypeStruct((n, _FAUG), jnp.bfloat16),
            jax.ShapeDtypeStruct((n, 1), jnp.float32),
            jax.ShapeDtypeStruct((1, n), jnp.float32),
            jax.ShapeDtypeStruct((1, 1), jnp.float32),
        ],
        compiler_params=pltpu.CompilerParams(
            dimension_semantics=("arbitrary",)),
    )(x, Wv, bv.reshape(1, f), wq, bq.reshape(1, 1), wk, bk.reshape(1, 1))

    bi = min(_BI, n)
    bj = min(_BJ, n)
    ni = n // bi
    nj = n // bj
    out = pl.pallas_call(
        functools.partial(_attn_kernel, ni=ni, f=f),
        grid=(nj, ni),
        in_specs=[
            pl.BlockSpec((bi, bj), lambda j, i: (i, j)),
            pl.BlockSpec((bi, 1), lambda j, i: (i, 0)),
            pl.BlockSpec((1, bj), lambda j, i: (0, j)),
            pl.BlockSpec((bi, _FAUG), lambda j, i: (i, 0)),
            pl.BlockSpec((1, 1), lambda j, i: (0, 0)),
        ],
        out_specs=pl.BlockSpec((bj, f), lambda j, i: (j, 0)),
        out_shape=jax.ShapeDtypeStruct((n, f), jnp.float32),
        scratch_shapes=[
            pltpu.VMEM((bj, _FAUG), jnp.float32),
        ],
        compiler_params=pltpu.CompilerParams(
            dimension_semantics=("parallel", "arbitrary")),
    )(adj, el, er, xvb, elmax)
    return out


# speedup vs baseline: 1.3635x; 1.0349x over previous
import functools

import jax
import jax.numpy as jnp
from jax.experimental import pallas as pl
from jax.experimental.pallas import tpu as pltpu

_LOG2E = 1.4426950408889634
_FAUG = 384
_BI = 512
_BJ = 4096


def _proj_kernel(x_ref, wv_ref, bv_ref, wq_ref, bq_ref, wk_ref, bk_ref,
                 xvb_ref, el_ref, er_ref, elmax_ref):
    i = pl.program_id(0)
    xv = jax.lax.dot_general(
        x_ref[...], wv_ref[...], (((1,), (1,)), ((), ())),
        preferred_element_type=jnp.float32) + bv_ref[...]
    xvb_ref[:, :xv.shape[1]] = xv.astype(jnp.bfloat16)
    lane = jax.lax.broadcasted_iota(
        jnp.int32, (xv.shape[0], _FAUG - xv.shape[1]), 1)
    xvb_ref[:, xv.shape[1]:] = (lane == 0).astype(jnp.bfloat16)
    el = jnp.sum(xv * wq_ref[...], axis=1, keepdims=True) + bq_ref[0, 0]
    el_ref[...] = el
    er_col = jnp.sum(xv * wk_ref[...], axis=1, keepdims=True) + bk_ref[0, 0]
    er_ref[...] = er_col.T
    bmax = jnp.max(el, keepdims=True)

    @pl.when(i == 0)
    def _():
        elmax_ref[...] = bmax

    @pl.when(i > 0)
    def _():
        elmax_ref[...] = jnp.maximum(elmax_ref[...], bmax)


def _attn_kernel(adj_ref, el_ref, er_ref, xvb_ref, elmax_ref, out_ref,
                 acc_ref, *, ni, f):
    i = pl.program_id(1)

    @pl.when(i == 0)
    def _():
        acc_ref[...] = jnp.zeros_like(acc_ref)

    el = el_ref[...]
    er = er_ref[...]
    mtop = elmax_ref[...] + er
    mj = jnp.maximum(mtop, 0.2 * mtop)
    a = (er - mj) * _LOG2E
    b = (0.2 * er - mj) * _LOG2E
    elc = el * _LOG2E
    elc2 = elc * 0.2
    z = jnp.maximum(elc + a, elc2 + b)
    p = (jnp.exp2(z) * adj_ref[...].astype(jnp.float32)).astype(jnp.bfloat16)
    acc_ref[...] += jax.lax.dot_general(
        p, xvb_ref[...], (((0,), (0,)), ((), ())),
        preferred_element_type=jnp.float32)

    @pl.when(i == ni - 1)
    def _():
        d = jnp.maximum(acc_ref[:, f:f + 1], 1e-20)
        out_ref[...] = acc_ref[:, :f] * (1.0 / d)


def kernel(x, adj, Wv, bv, wq, bq, wk, bk):
    n, _ = x.shape
    f = Wv.shape[0]

    bi1 = min(512, n)
    ni1 = n // bi1
    xvb, el, er, elmax = pl.pallas_call(
        _proj_kernel,
        grid=(ni1,),
        in_specs=[
            pl.BlockSpec((bi1, x.shape[1]), lambda i: (i, 0)),
            pl.BlockSpec(Wv.shape, lambda i: (0, 0)),
            pl.BlockSpec((1, f), lambda i: (0, 0)),
            pl.BlockSpec((1, f), lambda i: (0, 0)),
            pl.BlockSpec((1, 1), lambda i: (0, 0)),
            pl.BlockSpec((1, f), lambda i: (0, 0)),
            pl.BlockSpec((1, 1), lambda i: (0, 0)),
        ],
        out_specs=[
            pl.BlockSpec((bi1, _FAUG), lambda i: (i, 0)),
            pl.BlockSpec((bi1, 1), lambda i: (i, 0)),
            pl.BlockSpec((1, bi1), lambda i: (0, i)),
            pl.BlockSpec((1, 1), lambda i: (0, 0)),
        ],
        out_shape=[
            jax.ShapeDtypeStruct((n, _FAUG), jnp.bfloat16),
            jax.ShapeDtypeStruct((n, 1), jnp.float32),
            jax.ShapeDtypeStruct((1, n), jnp.float32),
            jax.ShapeDtypeStruct((1, 1), jnp.float32),
        ],
        compiler_params=pltpu.CompilerParams(
            dimension_semantics=("arbitrary",)),
    )(x, Wv, bv.reshape(1, f), wq, bq.reshape(1, 1), wk, bk.reshape(1, 1))

    bi = min(_BI, n)
    bj = min(_BJ, n)
    ni = n // bi
    nj = n // bj
    out = pl.pallas_call(
        functools.partial(_attn_kernel, ni=ni, f=f),
        grid=(nj, ni),
        in_specs=[
            pl.BlockSpec((bi, bj), lambda j, i: (i, j)),
            pl.BlockSpec((bi, 1), lambda j, i: (i, 0)),
            pl.BlockSpec((1, bj), lambda j, i: (0, j)),
            pl.BlockSpec((bi, _FAUG), lambda j, i: (i, 0)),
            pl.BlockSpec((1, 1), lambda j, i: (0, 0)),
        ],
        out_specs=pl.BlockSpec((bj, f), lambda j, i: (j, 0)),
        out_shape=jax.ShapeDtypeStruct((n, f), jnp.float32),
        scratch_shapes=[
            pltpu.VMEM((bj, _FAUG), jnp.float32),
        ],
        compiler_params=pltpu.CompilerParams(
            dimension_semantics=("parallel", "arbitrary")),
    )(adj, el, er, xvb, elmax)
    return out
